# baseline (speedup 1.0000x reference)
import functools

import jax
import jax.numpy as jnp
from jax import lax
from jax.experimental import pallas as pl
from jax.experimental.pallas import tpu as pltpu
from jax.experimental.pallas import tpu_sc as plsc

_N = 10000
_D = 128
_E = 320000
_NC = 2
_NS = 16
_NW = _NC * _NS
_CH = 80
_RB = 1000


def _make_sc_agg(n, d, e, gather=True):
  ept = e // _NW
  nchunk = ept // _CH
  nbuf = 3
  ngroup = nchunk // nbuf
  nrem = nchunk % nbuf
  rs = (n // _NS) & ~7
  tail_off = rs * _NS
  tail_sz = n - tail_off
  mesh = plsc.VectorSubcoreMesh(core_axis_name="c", subcore_axis_name="s")

  scratch = [
      pltpu.VMEM((nbuf, _CH), jnp.int32),
      pltpu.VMEM((nbuf, _CH, d), jnp.float32),
      pltpu.VMEM_SHARED((n, d), jnp.float32),
      [pltpu.SemaphoreType.DMA] * nbuf,
      [pltpu.SemaphoreType.DMA] * nbuf,
      [pltpu.SemaphoreType.DMA] * nbuf,
      pltpu.SemaphoreType.DMA,
  ]
  if gather:
    scratch.insert(0, pltpu.VMEM((ept,), jnp.int32))

  @functools.partial(
      pl.kernel,
      out_type=jax.ShapeDtypeStruct((_NC, n, d), jnp.float32),
      mesh=mesh,
      scratch_types=scratch,
  )
  def k(*refs):
    if gather:
      (g_hbm, src_hbm, dst_hbm, zero_hbm, out_hbm, sidx, dstage, rows, acc,
       gsem, ssem, dsem, zsem) = refs
    else:
      (dst_hbm, zero_hbm, out_hbm, dstage, rows, acc, gsem, ssem, dsem,
       zsem) = refs
    c = lax.axis_index("c")
    s = lax.axis_index("s")
    wid = s * _NC + c

    roff = s * rs
    if gather:
      pltpu.async_copy(src_hbm.at[pl.ds(wid * ept, ept)], sidx, gsem[0])
    else:
      nd16 = d // 16
      one = jnp.full((16,), 1.0, jnp.float32)

      def ob(i, carry):
        b = i // (_CH * nd16)
        rem = i % (_CH * nd16)
        rows[b, rem // nd16, pl.ds((rem % nd16) * 16, 16)] = one
        return carry

      lax.fori_loop(0, nbuf * _CH * nd16, ob, 0)
    pltpu.async_copy(zero_hbm.at[pl.ds(0, rs)], acc.at[pl.ds(roff, rs)], zsem)

    @pl.when(s == 0)
    def _():
      pltpu.async_copy(
          zero_hbm.at[pl.ds(0, tail_sz)], acc.at[pl.ds(tail_off, tail_sz)], ssem[0]
      )
      pltpu.make_async_copy(
          zero_hbm.at[pl.ds(0, tail_sz)], acc.at[pl.ds(tail_off, tail_sz)], ssem[0]
      ).wait()

    if gather:
      pltpu.make_async_copy(src_hbm.at[pl.ds(wid * ept, ept)], sidx, gsem[0]).wait()
    pltpu.make_async_copy(
        zero_hbm.at[pl.ds(0, rs)], acc.at[pl.ds(roff, rs)], zsem
    ).wait()
    plsc.subcore_barrier()

    def didx_start(j, b):
      pltpu.async_copy(
          dst_hbm.at[pl.ds(wid * ept + j * _CH, _CH)], dstage.at[b], dsem[b]
      )

    def didx_wait(j, b):
      pltpu.make_async_copy(
          dst_hbm.at[pl.ds(wid * ept + j * _CH, _CH)], dstage.at[b], dsem[b]
      ).wait()

    def gather_start(j, b):
      if gather:
        pltpu.async_copy(g_hbm.at[sidx.at[pl.ds(j * _CH, _CH)]], rows.at[b], gsem[b])

    def gather_wait(j, b):
      if gather:
        pltpu.make_async_copy(
            g_hbm.at[sidx.at[pl.ds(j * _CH, _CH)]], rows.at[b], gsem[b]
        ).wait()

    def scat_start(b):
      pltpu.async_copy(rows.at[b], acc.at[dstage.at[b]], ssem[b], add=True)

    def scat_wait(b):
      pltpu.make_async_copy(rows.at[b], acc.at[dstage.at[b]], ssem[b]).wait()

    for b in range(nbuf):
      didx_start(b, b)
      gather_start(b, b)

    @pl.loop(0, ngroup)
    def _(grp):
      base = grp * nbuf
      for b in range(nbuf):
        j = base + b
        gather_wait(j, b)
        didx_wait(j, b)
        scat_start(b)
      for b in range(nbuf):
        j = base + b
        scat_wait(b)

        @pl.when(j + nbuf < nchunk)
        def _():
          didx_start(j + nbuf, b)
          gather_start(j + nbuf, b)

    for t in range(nrem):
      j = ngroup * nbuf + t
      gather_wait(j, t)
      didx_wait(j, t)
      scat_start(t)
      scat_wait(t)

    plsc.subcore_barrier()

    pltpu.sync_copy(acc.at[pl.ds(roff, rs)], out_hbm.at[c, pl.ds(roff, rs)])

    @pl.when(s == 0)
    def _():
      pltpu.sync_copy(
          acc.at[pl.ds(tail_off, tail_sz)], out_hbm.at[c, pl.ds(tail_off, tail_sz)]
      )

  return k


_agg128 = _make_sc_agg(_N, _D, _E)
_deg = _make_sc_agg(_N, _D, _E, gather=False)


def _mm_body(x_ref, w_ref, dis_ref, g_ref):
  g_ref[...] = (
      jnp.dot(x_ref[...], w_ref[...], preferred_element_type=jnp.float32)
      * dis_ref[...]
  )


_mm = pl.pallas_call(
    _mm_body,
    grid=(_N // _RB,),
    in_specs=[
        pl.BlockSpec((_RB, _D), lambda i: (i, 0)),
        pl.BlockSpec((_D, _D), lambda i: (0, 0)),
        pl.BlockSpec((_RB, 1), lambda i: (i, 0)),
    ],
    out_specs=pl.BlockSpec((_RB, _D), lambda i: (i, 0)),
    out_shape=jax.ShapeDtypeStruct((_N, _D), jnp.float32),
)


def _dis_body(p_ref, dis_ref):
  deg = p_ref[0, :, :1] + p_ref[1, :, :1] + 1.0
  dis_ref[...] = lax.rsqrt(deg)


_dis = pl.pallas_call(
    _dis_body,
    grid=(_N // _RB,),
    in_specs=[pl.BlockSpec((_NC, _RB, _D), lambda i: (0, i, 0))],
    out_specs=pl.BlockSpec((_RB, 1), lambda i: (i, 0)),
    out_shape=jax.ShapeDtypeStruct((_N, 1), jnp.float32),
)


def _make_epilogue(mode):
  def body(*refs):
    if mode == "res":
      p_ref, g_ref, dis_ref, b_ref, xres_ref, out_ref = refs
    else:
      p_ref, g_ref, dis_ref, b_ref, out_ref = refs
    pre = (p_ref[0] + p_ref[1] + g_ref[...]) * dis_ref[...] + b_ref[...]
    if mode == "none":
      out_ref[...] = pre
    else:
      y = jnp.where(pre > 0, pre, 0.01 * pre)
      if mode == "res":
        y = (xres_ref[...] + y) * 0.5
      out_ref[...] = y

  in_specs = [
      pl.BlockSpec((_NC, _RB, _D), lambda i: (0, i, 0)),
      pl.BlockSpec((_RB, _D), lambda i: (i, 0)),
      pl.BlockSpec((_RB, 1), lambda i: (i, 0)),
      pl.BlockSpec((1, _D), lambda i: (0, 0)),
  ]
  if mode == "res":
    in_specs.append(pl.BlockSpec((_RB, _D), lambda i: (i, 0)))
  return pl.pallas_call(
      body,
      grid=(_N // _RB,),
      in_specs=in_specs,
      out_specs=pl.BlockSpec((_RB, _D), lambda i: (i, 0)),
      out_shape=jax.ShapeDtypeStruct((_N, _D), jnp.float32),
  )


_ep_none = _make_epilogue("none")


def _make_fused(mode):
  def body(*refs):
    if mode == "res":
      p_ref, g_ref, dis_ref, b_ref, xres_ref, w_ref, y_ref, gn_ref = refs
    else:
      p_ref, g_ref, dis_ref, b_ref, w_ref, y_ref, gn_ref = refs
    pre = (p_ref[0] + p_ref[1] + g_ref[...]) * dis_ref[...] + b_ref[...]
    y = jnp.where(pre > 0, pre, 0.01 * pre)
    if mode == "res":
      y = (xres_ref[...] + y) * 0.5
    y_ref[...] = y
    gn_ref[...] = (
        jnp.dot(y, w_ref[...], preferred_element_type=jnp.float32)
        * dis_ref[...]
    )

  in_specs = [
      pl.BlockSpec((_NC, _RB, _D), lambda i: (0, i, 0)),
      pl.BlockSpec((_RB, _D), lambda i: (i, 0)),
      pl.BlockSpec((_RB, 1), lambda i: (i, 0)),
      pl.BlockSpec((1, _D), lambda i: (0, 0)),
  ]
  if mode == "res":
    in_specs.append(pl.BlockSpec((_RB, _D), lambda i: (i, 0)))
  in_specs.append(pl.BlockSpec((_D, _D), lambda i: (0, 0)))
  return pl.pallas_call(
      body,
      grid=(_N // _RB,),
      in_specs=in_specs,
      out_specs=[
          pl.BlockSpec((_RB, _D), lambda i: (i, 0)),
          pl.BlockSpec((_RB, _D), lambda i: (i, 0)),
      ],
      out_shape=[
          jax.ShapeDtypeStruct((_N, _D), jnp.float32),
          jax.ShapeDtypeStruct((_N, _D), jnp.float32),
      ],
  )


_fused_lrelu = _make_fused("lrelu")
_fused_res = _make_fused("res")


def kernel(inputs, edges, W1, b1, Wb1, bb1, Wb2, bb2, W2, b2):
  src = edges[0]
  dst = edges[1]
  zeros = jnp.zeros(((_N // _NS) & ~7, _D), jnp.float32)

  degp = _deg(dst, zeros)
  dis = _dis(degp)

  g = _mm(inputs, W1, dis)
  next_w = [Wb2[0]]
  for i in range(1, 6):
    next_w += [Wb1[i], Wb2[i]]
  next_w += [W2]

  part = _agg128(g, src, dst, zeros)
  x, g = _fused_lrelu(part, g, dis, b1.reshape(1, _D), Wb1[0])
  for i in range(6):
    part = _agg128(g, src, dst, zeros)
    _, g = _fused_lrelu(part, g, dis, bb1[i].reshape(1, _D), next_w[2 * i])
    part = _agg128(g, src, dst, zeros)
    x, g = _fused_res(part, g, dis, bb2[i].reshape(1, _D), x, next_w[2 * i + 1])
  part = _agg128(g, src, dst, zeros)
  x_out = _ep_none(part, g, dis, b2.reshape(1, _D))
  return (x_out, x)


# speedup vs baseline: 1.0707x; 1.0707x over previous
import functools

import jax
import jax.numpy as jnp
from jax import lax
from jax.experimental import pallas as pl
from jax.experimental.pallas import tpu as pltpu
from jax.experimental.pallas import tpu_sc as plsc

_N = 10000
_D = 128
_E = 320000
_NC = 2
_NS = 16
_NW = _NC * _NS
_CH = 40
_RB = 1000


def _make_sc_agg(n, d, e, gather=True):
  ept = e // _NW
  nchunk = ept // _CH
  nbuf = 6
  ngroup = nchunk // nbuf
  nrem = nchunk % nbuf
  rs = (n // _NS) & ~7
  tail_off = rs * _NS
  tail_sz = n - tail_off
  mesh = plsc.VectorSubcoreMesh(core_axis_name="c", subcore_axis_name="s")

  scratch = [
      pltpu.VMEM((nbuf, _CH), jnp.int32),
      pltpu.VMEM((nbuf, _CH, d), jnp.float32),
      pltpu.VMEM_SHARED((n, d), jnp.float32),
      [pltpu.SemaphoreType.DMA] * nbuf,
      [pltpu.SemaphoreType.DMA] * nbuf,
      [pltpu.SemaphoreType.DMA] * nbuf,
      pltpu.SemaphoreType.DMA,
  ]
  if gather:
    scratch.insert(0, pltpu.VMEM((ept,), jnp.int32))

  @functools.partial(
      pl.kernel,
      out_type=jax.ShapeDtypeStruct((_NC, n, d), jnp.float32),
      mesh=mesh,
      scratch_types=scratch,
  )
  def k(*refs):
    if gather:
      (g_hbm, src_hbm, dst_hbm, zero_hbm, out_hbm, sidx, dstage, rows, acc,
       gsem, ssem, dsem, zsem) = refs
    else:
      (dst_hbm, zero_hbm, out_hbm, dstage, rows, acc, gsem, ssem, dsem,
       zsem) = refs
    c = lax.axis_index("c")
    s = lax.axis_index("s")
    wid = s * _NC + c

    roff = s * rs
    if gather:
      pltpu.async_copy(src_hbm.at[pl.ds(wid * ept, ept)], sidx, gsem[0])
    else:
      nd16 = d // 16
      one = jnp.full((16,), 1.0, jnp.float32)

      def ob(i, carry):
        b = i // (_CH * nd16)
        rem = i % (_CH * nd16)
        rows[b, rem // nd16, pl.ds((rem % nd16) * 16, 16)] = one
        return carry

      lax.fori_loop(0, nbuf * _CH * nd16, ob, 0)
    pltpu.async_copy(zero_hbm.at[pl.ds(0, rs)], acc.at[pl.ds(roff, rs)], zsem)

    @pl.when(s == 0)
    def _():
      pltpu.async_copy(
          zero_hbm.at[pl.ds(0, tail_sz)], acc.at[pl.ds(tail_off, tail_sz)], ssem[0]
      )
      pltpu.make_async_copy(
          zero_hbm.at[pl.ds(0, tail_sz)], acc.at[pl.ds(tail_off, tail_sz)], ssem[0]
      ).wait()

    if gather:
      pltpu.make_async_copy(src_hbm.at[pl.ds(wid * ept, ept)], sidx, gsem[0]).wait()
    pltpu.make_async_copy(
        zero_hbm.at[pl.ds(0, rs)], acc.at[pl.ds(roff, rs)], zsem
    ).wait()
    plsc.subcore_barrier()

    def didx_start(j, b):
      pltpu.async_copy(
          dst_hbm.at[pl.ds(wid * ept + j * _CH, _CH)], dstage.at[b], dsem[b]
      )

    def didx_wait(j, b):
      pltpu.make_async_copy(
          dst_hbm.at[pl.ds(wid * ept + j * _CH, _CH)], dstage.at[b], dsem[b]
      ).wait()

    def gather_start(j, b):
      if gather:
        pltpu.async_copy(g_hbm.at[sidx.at[pl.ds(j * _CH, _CH)]], rows.at[b], gsem[b])

    def gather_wait(j, b):
      if gather:
        pltpu.make_async_copy(
            g_hbm.at[sidx.at[pl.ds(j * _CH, _CH)]], rows.at[b], gsem[b]
        ).wait()

    def scat_start(b):
      pltpu.async_copy(rows.at[b], acc.at[dstage.at[b]], ssem[b], add=True)

    def scat_wait(b):
      pltpu.make_async_copy(rows.at[b], acc.at[dstage.at[b]], ssem[b]).wait()

    for b in range(nbuf):
      didx_start(b, b)
      gather_start(b, b)

    @pl.loop(0, ngroup)
    def _(grp):
      base = grp * nbuf
      for b in range(nbuf):
        j = base + b
        gather_wait(j, b)
        didx_wait(j, b)
        scat_start(b)
      for b in range(nbuf):
        j = base + b
        scat_wait(b)

        @pl.when(j + nbuf < nchunk)
        def _():
          didx_start(j + nbuf, b)
          gather_start(j + nbuf, b)

    for t in range(nrem):
      j = ngroup * nbuf + t
      gather_wait(j, t)
      didx_wait(j, t)
      scat_start(t)
      scat_wait(t)

    plsc.subcore_barrier()

    pltpu.sync_copy(acc.at[pl.ds(roff, rs)], out_hbm.at[c, pl.ds(roff, rs)])

    @pl.when(s == 0)
    def _():
      pltpu.sync_copy(
          acc.at[pl.ds(tail_off, tail_sz)], out_hbm.at[c, pl.ds(tail_off, tail_sz)]
      )

  return k


_agg128 = _make_sc_agg(_N, _D, _E)
_deg = _make_sc_agg(_N, _D, _E, gather=False)


def _mm_body(x_ref, w_ref, dis_ref, g_ref):
  g_ref[...] = (
      jnp.dot(x_ref[...], w_ref[...], preferred_element_type=jnp.float32)
      * dis_ref[...]
  )


_mm = pl.pallas_call(
    _mm_body,
    grid=(_N // _RB,),
    in_specs=[
        pl.BlockSpec((_RB, _D), lambda i: (i, 0)),
        pl.BlockSpec((_D, _D), lambda i: (0, 0)),
        pl.BlockSpec((_RB, 1), lambda i: (i, 0)),
    ],
    out_specs=pl.BlockSpec((_RB, _D), lambda i: (i, 0)),
    out_shape=jax.ShapeDtypeStruct((_N, _D), jnp.float32),
)


def _dis_body(p_ref, dis_ref):
  deg = p_ref[0, :, :1] + p_ref[1, :, :1] + 1.0
  dis_ref[...] = lax.rsqrt(deg)


_dis = pl.pallas_call(
    _dis_body,
    grid=(_N // _RB,),
    in_specs=[pl.BlockSpec((_NC, _RB, _D), lambda i: (0, i, 0))],
    out_specs=pl.BlockSpec((_RB, 1), lambda i: (i, 0)),
    out_shape=jax.ShapeDtypeStruct((_N, 1), jnp.float32),
)


def _make_epilogue(mode):
  def body(*refs):
    if mode == "res":
      p_ref, g_ref, dis_ref, b_ref, xres_ref, out_ref = refs
    else:
      p_ref, g_ref, dis_ref, b_ref, out_ref = refs
    pre = (p_ref[0] + p_ref[1] + g_ref[...]) * dis_ref[...] + b_ref[...]
    if mode == "none":
      out_ref[...] = pre
    else:
      y = jnp.where(pre > 0, pre, 0.01 * pre)
      if mode == "res":
        y = (xres_ref[...] + y) * 0.5
      out_ref[...] = y

  in_specs = [
      pl.BlockSpec((_NC, _RB, _D), lambda i: (0, i, 0)),
      pl.BlockSpec((_RB, _D), lambda i: (i, 0)),
      pl.BlockSpec((_RB, 1), lambda i: (i, 0)),
      pl.BlockSpec((1, _D), lambda i: (0, 0)),
  ]
  if mode == "res":
    in_specs.append(pl.BlockSpec((_RB, _D), lambda i: (i, 0)))
  return pl.pallas_call(
      body,
      grid=(_N // _RB,),
      in_specs=in_specs,
      out_specs=pl.BlockSpec((_RB, _D), lambda i: (i, 0)),
      out_shape=jax.ShapeDtypeStruct((_N, _D), jnp.float32),
  )


_ep_none = _make_epilogue("none")


def _make_fused(mode):
  def body(*refs):
    if mode == "res":
      p_ref, g_ref, dis_ref, b_ref, xres_ref, w_ref, y_ref, gn_ref = refs
    else:
      p_ref, g_ref, dis_ref, b_ref, w_ref, y_ref, gn_ref = refs
    pre = (p_ref[0] + p_ref[1] + g_ref[...]) * dis_ref[...] + b_ref[...]
    y = jnp.where(pre > 0, pre, 0.01 * pre)
    if mode == "res":
      y = (xres_ref[...] + y) * 0.5
    y_ref[...] = y
    gn_ref[...] = (
        jnp.dot(y, w_ref[...], preferred_element_type=jnp.float32)
        * dis_ref[...]
    )

  in_specs = [
      pl.BlockSpec((_NC, _RB, _D), lambda i: (0, i, 0)),
      pl.BlockSpec((_RB, _D), lambda i: (i, 0)),
      pl.BlockSpec((_RB, 1), lambda i: (i, 0)),
      pl.BlockSpec((1, _D), lambda i: (0, 0)),
  ]
  if mode == "res":
    in_specs.append(pl.BlockSpec((_RB, _D), lambda i: (i, 0)))
  in_specs.append(pl.BlockSpec((_D, _D), lambda i: (0, 0)))
  return pl.pallas_call(
      body,
      grid=(_N // _RB,),
      in_specs=in_specs,
      out_specs=[
          pl.BlockSpec((_RB, _D), lambda i: (i, 0)),
          pl.BlockSpec((_RB, _D), lambda i: (i, 0)),
      ],
      out_shape=[
          jax.ShapeDtypeStruct((_N, _D), jnp.float32),
          jax.ShapeDtypeStruct((_N, _D), jnp.float32),
      ],
  )


_fused_lrelu = _make_fused("lrelu")
_fused_res = _make_fused("res")


def kernel(inputs, edges, W1, b1, Wb1, bb1, Wb2, bb2, W2, b2):
  src = edges[0]
  dst = edges[1]
  zeros = jnp.zeros(((_N // _NS) & ~7, _D), jnp.float32)

  degp = _deg(dst, zeros)
  dis = _dis(degp)

  g = _mm(inputs, W1, dis)
  next_w = [Wb2[0]]
  for i in range(1, 6):
    next_w += [Wb1[i], Wb2[i]]
  next_w += [W2]

  part = _agg128(g, src, dst, zeros)
  x, g = _fused_lrelu(part, g, dis, b1.reshape(1, _D), Wb1[0])
  for i in range(6):
    part = _agg128(g, src, dst, zeros)
    _, g = _fused_lrelu(part, g, dis, bb1[i].reshape(1, _D), next_w[2 * i])
    part = _agg128(g, src, dst, zeros)
    x, g = _fused_res(part, g, dis, bb2[i].reshape(1, _D), x, next_w[2 * i + 1])
  part = _agg128(g, src, dst, zeros)
  x_out = _ep_none(part, g, dis, b2.reshape(1, _D))
  return (x_out, x)
